# Initial kernel scaffold; baseline (speedup 1.0000x reference)
import jax
import jax.numpy as jnp
from jax.experimental import pallas as pl


def kernel(z_loga):
    raise NotImplementedError("write your pallas kernel here")


# speedup vs baseline: 34.8619x; 34.8619x over previous
import functools

import jax
import jax.numpy as jnp
from jax.experimental import pallas as pl
from jax.experimental.pallas import tpu as pltpu

_TEMPERATURE = 2.0 / 3.0
_MAGIC = 0.8
_ROWS = 32
_COLS = 32768
_NUM_ZEROS = _COLS - 16384
_ONE_BITS = 0x3F800000


def _body(z_ref, o_ref):
    z = z_ref[...]
    s = jax.nn.sigmoid(z / _TEMPERATURE * _MAGIC)
    keys = jax.lax.bitcast_convert_type(s, jnp.int32)
    rows = z.shape[0]
    k = _NUM_ZEROS

    lo = jnp.zeros((rows, 1), jnp.int32)
    hi = jnp.full((rows, 1), _ONE_BITS, jnp.int32)

    def val_it(_, carry):
        lo, hi = carry
        mid = lo + (hi - lo) // 2
        cnt = jnp.sum((keys <= mid).astype(jnp.int32), axis=1, keepdims=True)
        ge = cnt >= k
        return jnp.where(ge, lo, mid + 1), jnp.where(ge, mid, hi)

    tbits, _ = jax.lax.fori_loop(0, 31, val_it, (lo, hi))

    c_lt = jnp.sum((keys < tbits).astype(jnp.int32), axis=1, keepdims=True)
    m = k - c_lt
    eq = keys == tbits
    idx = jax.lax.broadcasted_iota(jnp.int32, z.shape, 1)

    lo2 = jnp.zeros((rows, 1), jnp.int32)
    hi2 = jnp.full((rows, 1), _COLS, jnp.int32)

    def idx_it(_, carry):
        lo, hi = carry
        mid = lo + (hi - lo) // 2
        cnt = jnp.sum((eq & (idx < mid)).astype(jnp.int32), axis=1,
                      keepdims=True)
        ge = cnt >= m
        return jnp.where(ge, lo, mid + 1), jnp.where(ge, mid, hi)

    icut, _ = jax.lax.fori_loop(0, 16, idx_it, (lo2, hi2))

    zero_mask = (keys < tbits) | (eq & (idx < icut))
    o_ref[...] = jnp.where(zero_mask, 0.0, s)


@jax.jit
def kernel(z_loga):
    block_rows = 8
    return pl.pallas_call(
        _body,
        grid=(_ROWS // block_rows,),
        in_specs=[pl.BlockSpec((block_rows, _COLS), lambda i: (i, 0))],
        out_specs=pl.BlockSpec((block_rows, _COLS), lambda i: (i, 0)),
        out_shape=jax.ShapeDtypeStruct((_ROWS, _COLS), jnp.float32),
    )(z_loga)
